# baseline (speedup 1.0000x reference)
import functools

import jax
import jax.numpy as jnp
from jax import lax
from jax.experimental import pallas as pl
from jax.experimental.pallas import tpu as pltpu
from jax.experimental.pallas import tpu_sc as plsc

NN = 10000
NE = 160000
CH = 256
KV = 27
EPSV = 1e-6

NP_ = 10240
RPT = NP_ // 32
GB = 88
EC = 32000
NCH = NE // EC
RING = 4096
FB = 512
ESTRIDE = NE + 4 * FB


def _mm27_body(x_ref, w_ref, o_ref):
    xb = x_ref[...].astype(jnp.bfloat16)
    for k in range(KV):
        o_ref[k] = jnp.dot(xb, w_ref[k], preferred_element_type=jnp.float32)


def _ln_relu_mm27_body(h_ref, g_ref, b_ref, w_ref, o_ref):
    h = h_ref[...]
    mu = jnp.mean(h, axis=1, keepdims=True)
    d = h - mu
    var = jnp.mean(d * d, axis=1, keepdims=True)
    y = d * lax.rsqrt(var + EPSV) * g_ref[...] + b_ref[...]
    y = jnp.maximum(y, 0.0).astype(jnp.bfloat16)
    for k in range(KV):
        o_ref[k] = jnp.dot(y, w_ref[k], preferred_element_type=jnp.float32)


def _final_body(h_ref, g_ref, b_ref, x_ref, o_ref):
    h = h_ref[...]
    mu = jnp.mean(h, axis=1, keepdims=True)
    d = h - mu
    var = jnp.mean(d * d, axis=1, keepdims=True)
    y = d * lax.rsqrt(var + EPSV) * g_ref[...] + b_ref[...]
    o_ref[...] = jnp.maximum(y + x_ref[...], 0.0)


_BN = 256


def _mm27(x_pad, W):
    return pl.pallas_call(
        _mm27_body,
        grid=(NP_ // _BN,),
        in_specs=[
            pl.BlockSpec((_BN, CH), lambda i: (i, 0)),
            pl.BlockSpec((KV, CH, CH), lambda i: (0, 0, 0)),
        ],
        out_specs=pl.BlockSpec((KV, _BN, CH), lambda i: (0, i, 0)),
        out_shape=jax.ShapeDtypeStruct((KV, NP_, CH), jnp.float32),
    )(x_pad, W)


def _ln_relu_mm27(h_pad, g, b, W):
    return pl.pallas_call(
        _ln_relu_mm27_body,
        grid=(NP_ // _BN,),
        in_specs=[
            pl.BlockSpec((_BN, CH), lambda i: (i, 0)),
            pl.BlockSpec((1, CH), lambda i: (0, 0)),
            pl.BlockSpec((1, CH), lambda i: (0, 0)),
            pl.BlockSpec((KV, CH, CH), lambda i: (0, 0, 0)),
        ],
        out_specs=pl.BlockSpec((KV, _BN, CH), lambda i: (0, i, 0)),
        out_shape=jax.ShapeDtypeStruct((KV, NP_, CH), jnp.float32),
    )(h_pad, g, b, W)


_FBN = 400


def _final(h, g, b, x):
    return pl.pallas_call(
        _final_body,
        grid=(NN // _FBN,),
        in_specs=[
            pl.BlockSpec((_FBN, CH), lambda i: (i, 0)),
            pl.BlockSpec((1, CH), lambda i: (0, 0)),
            pl.BlockSpec((1, CH), lambda i: (0, 0)),
            pl.BlockSpec((_FBN, CH), lambda i: (i, 0)),
        ],
        out_specs=pl.BlockSpec((_FBN, CH), lambda i: (i, 0)),
        out_shape=jax.ShapeDtypeStruct((NN, CH), jnp.float32),
    )(h, g, b, x)


_sc_mesh = plsc.VectorSubcoreMesh(core_axis_name="c", subcore_axis_name="s")
_sc_params = pltpu.CompilerParams(needs_layout_passes=False)


@functools.partial(
    pl.kernel,
    mesh=_sc_mesh,
    compiler_params=_sc_params,
    out_type=(
        jax.ShapeDtypeStruct((32 * ESTRIDE,), jnp.int32),
        jax.ShapeDtypeStruct((32 * ESTRIDE,), jnp.int32),
        jax.ShapeDtypeStruct((32, 16), jnp.int32),
    ),
    scratch_types=[
        pltpu.VMEM((EC,), jnp.int32),
        pltpu.VMEM((EC,), jnp.int32),
        pltpu.VMEM((EC,), jnp.int32),
        pltpu.VMEM((RING,), jnp.int32),
        pltpu.VMEM((RING,), jnp.int32),
        pltpu.VMEM((16,), jnp.int32),
        pltpu.SemaphoreType.DMA,
        pltpu.SemaphoreType.DMA,
        pltpu.SemaphoreType.DMA,
    ],
)
def _sc_route(src_hbm, dst_hbm, kk_hbm, flat_hbm, loc_hbm, cnt_hbm,
              srcv, dstv, k_v, rflat, rloc, cbuf, semx, semy, semz):
    c = lax.axis_index("c")
    s = lax.axis_index("s")
    tid = s * 2 + c
    lbase = tid * ESTRIDE

    zero16 = jnp.zeros((16,), jnp.int32)

    def init_body(i, carry):
        rflat[pl.ds(i * 16, 16)] = zero16
        rloc[pl.ds(i * 16, 16)] = zero16
        return carry

    lax.fori_loop(0, RING // 16, init_body, jnp.int32(0))

    lo_v = jnp.full((16,), tid * RPT, jnp.int32)
    rpt_v = jnp.full((16,), RPT, jnp.int32)
    np_v = jnp.full((16,), NP_, jnp.int32)
    one_v = jnp.full((16,), 1, jnp.int32)
    rmask_v = jnp.full((16,), RING - 1, jnp.int32)

    idx15 = jnp.full((16,), 15, jnp.int32)

    def comp_body(i, carry):
        cnt_v, fl = carry
        ga = []
        for h in range(2):
            dv = dstv[pl.ds(i * 32 + h * 16, 16)]
            sv = srcv[pl.ds(i * 32 + h * 16, 16)]
            kv = k_v[pl.ds(i * 32 + h * 16, 16)]
            loc = dv - lo_v
            m = (loc >= zero16) & (loc < rpt_v)
            flat = kv * np_v + sv
            mi = m.astype(jnp.int32)
            scan = plsc.cumsum(mi)
            ga.append((loc, m, flat, scan))
        loc0, m0, flat0, scan0 = ga[0]
        loc1, m1, flat1, scan1 = ga[1]
        tot0 = lax.gather(
            scan0, idx15[:, None],
            lax.GatherDimensionNumbers(offset_dims=(),
                                       collapsed_slice_dims=(0,),
                                       start_index_map=(0,)),
            (1,), mode=lax.GatherScatterMode.PROMISE_IN_BOUNDS)
        tot1 = lax.gather(
            scan1, idx15[:, None],
            lax.GatherDimensionNumbers(offset_dims=(),
                                       collapsed_slice_dims=(0,),
                                       start_index_map=(0,)),
            (1,), mode=lax.GatherScatterMode.PROMISE_IN_BOUNDS)
        pos0 = (cnt_v + scan0 - one_v) & rmask_v
        base1 = cnt_v + tot0
        pos1 = (base1 + scan1 - one_v) & rmask_v
        plsc.store_scatter(rflat, [pos0], flat0, mask=m0)
        plsc.store_scatter(rloc, [pos0], loc0, mask=m0)
        plsc.store_scatter(rflat, [pos1], flat1, mask=m1)
        plsc.store_scatter(rloc, [pos1], loc1, mask=m1)
        return base1 + tot1, fl

    def flush_body(fi, fl):
        rpos = pl.multiple_of(fl % RING, FB)
        fl8 = pl.multiple_of(lbase + fl, FB)
        pltpu.sync_copy(rflat.at[pl.ds(rpos, FB)],
                        flat_hbm.at[pl.ds(fl8, FB)])
        pltpu.sync_copy(rloc.at[pl.ds(rpos, FB)],
                        loc_hbm.at[pl.ds(fl8, FB)])
        return fl + FB

    def chunk_body(ci, carry):
        cnt_v, fl = carry
        cbase = pl.multiple_of(ci * EC, 8)
        d1 = pltpu.async_copy(src_hbm.at[pl.ds(cbase, EC)], srcv, semx)
        d2 = pltpu.async_copy(dst_hbm.at[pl.ds(cbase, EC)], dstv, semy)
        d3 = pltpu.async_copy(kk_hbm.at[pl.ds(cbase, EC)], k_v, semz)
        d1.wait()
        d2.wait()
        d3.wait()
        cnt_v, fl = lax.fori_loop(0, EC // 32, comp_body, (cnt_v, fl))
        nf = (jnp.max(cnt_v) - fl) // FB
        fl = lax.fori_loop(0, nf, flush_body, fl)
        return cnt_v, fl

    cnt_v, fl = lax.fori_loop(0, NCH, chunk_body,
                              (jnp.zeros((16,), jnp.int32), jnp.int32(0)))
    cnt = jnp.max(cnt_v)
    for _ in range(4):
        fl = flush_body(0, fl)

    cbuf[...] = jnp.full((16,), cnt, jnp.int32)
    pltpu.sync_copy(cbuf, cnt_hbm.at[tid])


@functools.partial(
    pl.kernel,
    mesh=_sc_mesh,
    compiler_params=_sc_params,
    out_type=jax.ShapeDtypeStruct((NP_, CH), jnp.float32),
    scratch_types=[
        pltpu.VMEM((RPT, CH), jnp.float32),
        pltpu.VMEM((GB, CH), jnp.float32),
        pltpu.VMEM((GB, CH), jnp.float32),
        pltpu.VMEM((16 * GB,), jnp.int32),
        pltpu.VMEM((16 * GB,), jnp.int32),
        pltpu.VMEM((16,), jnp.int32),
        pltpu.SemaphoreType.DMA,
        pltpu.SemaphoreType.DMA,
    ],
)
def _sc_conv(h_tab, flat_hbm, loc_hbm, cnt_hbm, zero_hbm, out_hbm,
             acc, stage0, stage1, lflat, lloc, cbuf, sem0, sem1):
    c = lax.axis_index("c")
    s = lax.axis_index("s")
    tid = s * 2 + c
    lbase = tid * ESTRIDE
    lchunk = 16 * GB

    pltpu.sync_copy(zero_hbm, acc)
    pltpu.sync_copy(cnt_hbm.at[tid], cbuf)
    cnt = jnp.max(cbuf[...])
    nb = (cnt + GB - 1) // GB
    nc = (nb + 15) // 16

    iota16 = lax.iota(jnp.int32, 16)
    colb = [iota16 + jnp.full((16,), g * 16, jnp.int32)
            for g in range(CH // 16)]
    _dn = lax.GatherDimensionNumbers(offset_dims=(),
                                     collapsed_slice_dims=(0,),
                                     start_index_map=(0,))

    def chunk_body(ch, carry):
        chb = pl.multiple_of(lbase + ch * lchunk, 8)
        dl1 = pltpu.async_copy(flat_hbm.at[pl.ds(chb, lchunk)], lflat, sem0)
        dl2 = pltpu.async_copy(loc_hbm.at[pl.ds(chb, lchunk)], lloc, sem1)
        dl1.wait()
        dl2.wait()
        base_b = ch * 16

        def issue(jj, st, sm):
            cond = base_b + jj < nb

            @pl.when(cond)
            def _():
                pltpu.async_copy(h_tab.at[lflat.at[pl.ds(jj * GB, GB)]],
                                 st, sm)

            return cond

        def drain(jj, st):
            gj = base_b + jj
            nvalid = jnp.clip(cnt - gj * GB, 0, GB)

            def one(r):
                lv = lloc[pl.ds(jj * GB + (r // 16) * 16, 16)]
                lane_v = jnp.full((16,), r % 16, jnp.int32)
                locsplat = lax.gather(
                    lv, lane_v[:, None], _dn, (1,),
                    mode=lax.GatherScatterMode.PROMISE_IN_BOUNDS)
                for g in range(CH // 16):
                    plsc.addupdate_scatter(acc, [locsplat, colb[g]],
                                           st[r, pl.ds(g * 16, 16)])

            def dbody2(q, cc):
                one(q * 2)
                one(q * 2 + 1)
                return cc

            lax.fori_loop(0, nvalid // 2, dbody2, jnp.int32(0))

            @pl.when(nvalid % 2 == 1)
            def _():
                one(nvalid - 1)

        slots = [(stage0, sem0), (stage1, sem1)]
        conds = {0: issue(0, *slots[0])}
        for jj in range(16):
            st, sm = slots[jj % 2]
            if jj + 1 < 16:
                conds[jj + 1] = issue(jj + 1, *slots[(jj + 1) % 2])

            @pl.when(conds[jj])
            def _(st=st, sm=sm):
                pltpu.make_async_copy(h_tab.at[pl.ds(0, GB)], st, sm).wait()

            drain(jj, st)
        return carry

    lax.fori_loop(0, nc, chunk_body, jnp.int32(0))

    pltpu.sync_copy(acc, out_hbm.at[pl.ds(tid * RPT, RPT)])


def kernel(x, edge_index, edge_kernel, W1, g1, b1, W2, g2, b2):
    src = edge_index[0].astype(jnp.int32)
    dst = edge_index[1].astype(jnp.int32)
    kk = edge_kernel.astype(jnp.int32)
    x_pad = jnp.pad(x, ((0, NP_ - NN), (0, 0)))
    W1b = W1.astype(jnp.bfloat16)
    W2b = W2.astype(jnp.bfloat16)
    zeros = jnp.zeros((RPT, CH), jnp.float32)
    g1r = g1.reshape(1, CH)
    b1r = b1.reshape(1, CH)
    g2r = g2.reshape(1, CH)
    b2r = b2.reshape(1, CH)

    flat_l, loc_l, cnts = _sc_route(src, dst, kk)

    H1 = _mm27(x_pad, W1b).reshape(KV * NP_, CH)
    h1 = _sc_conv(H1, flat_l, loc_l, cnts, zeros)
    H2 = _ln_relu_mm27(h1, g1r, b1r, W2b).reshape(KV * NP_, CH)
    h2 = _sc_conv(H2, flat_l, loc_l, cnts, zeros)
    return _final(h2[:NN], g2r, b2r, x)


# speedup vs baseline: 1.0045x; 1.0045x over previous
import functools

import jax
import jax.numpy as jnp
from jax import lax
from jax.experimental import pallas as pl
from jax.experimental.pallas import tpu as pltpu
from jax.experimental.pallas import tpu_sc as plsc

NN = 10000
NE = 160000
CH = 256
KV = 27
EPSV = 1e-6

NP_ = 10240
RPT = NP_ // 32
GB = 88
EC = 16000
NCH = NE // EC
RING = 4096
FB = 512
ESTRIDE = NE + 4 * FB


def _mm27_body(x_ref, w_ref, o_ref):
    xb = x_ref[...].astype(jnp.bfloat16)
    for k in range(KV):
        o_ref[k] = jnp.dot(xb, w_ref[k], preferred_element_type=jnp.float32)


def _ln_relu_mm27_body(h_ref, g_ref, b_ref, w_ref, o_ref):
    h = h_ref[...]
    mu = jnp.mean(h, axis=1, keepdims=True)
    d = h - mu
    var = jnp.mean(d * d, axis=1, keepdims=True)
    y = d * lax.rsqrt(var + EPSV) * g_ref[...] + b_ref[...]
    y = jnp.maximum(y, 0.0).astype(jnp.bfloat16)
    for k in range(KV):
        o_ref[k] = jnp.dot(y, w_ref[k], preferred_element_type=jnp.float32)


def _final_body(h_ref, g_ref, b_ref, x_ref, o_ref):
    h = h_ref[...]
    mu = jnp.mean(h, axis=1, keepdims=True)
    d = h - mu
    var = jnp.mean(d * d, axis=1, keepdims=True)
    y = d * lax.rsqrt(var + EPSV) * g_ref[...] + b_ref[...]
    o_ref[...] = jnp.maximum(y + x_ref[...], 0.0)


_BN = 256


def _mm27(x_pad, W):
    return pl.pallas_call(
        _mm27_body,
        grid=(NP_ // _BN,),
        in_specs=[
            pl.BlockSpec((_BN, CH), lambda i: (i, 0)),
            pl.BlockSpec((KV, CH, CH), lambda i: (0, 0, 0)),
        ],
        out_specs=pl.BlockSpec((KV, _BN, CH), lambda i: (0, i, 0)),
        out_shape=jax.ShapeDtypeStruct((KV, NP_, CH), jnp.float32),
    )(x_pad, W)


def _ln_relu_mm27(h_pad, g, b, W):
    return pl.pallas_call(
        _ln_relu_mm27_body,
        grid=(NP_ // _BN,),
        in_specs=[
            pl.BlockSpec((_BN, CH), lambda i: (i, 0)),
            pl.BlockSpec((1, CH), lambda i: (0, 0)),
            pl.BlockSpec((1, CH), lambda i: (0, 0)),
            pl.BlockSpec((KV, CH, CH), lambda i: (0, 0, 0)),
        ],
        out_specs=pl.BlockSpec((KV, _BN, CH), lambda i: (0, i, 0)),
        out_shape=jax.ShapeDtypeStruct((KV, NP_, CH), jnp.float32),
    )(h_pad, g, b, W)


_FBN = 400


def _final(h, g, b, x):
    return pl.pallas_call(
        _final_body,
        grid=(NN // _FBN,),
        in_specs=[
            pl.BlockSpec((_FBN, CH), lambda i: (i, 0)),
            pl.BlockSpec((1, CH), lambda i: (0, 0)),
            pl.BlockSpec((1, CH), lambda i: (0, 0)),
            pl.BlockSpec((_FBN, CH), lambda i: (i, 0)),
        ],
        out_specs=pl.BlockSpec((_FBN, CH), lambda i: (i, 0)),
        out_shape=jax.ShapeDtypeStruct((NN, CH), jnp.float32),
    )(h, g, b, x)


_sc_mesh = plsc.VectorSubcoreMesh(core_axis_name="c", subcore_axis_name="s")
_sc_params = pltpu.CompilerParams(needs_layout_passes=False)


@functools.partial(
    pl.kernel,
    mesh=_sc_mesh,
    compiler_params=_sc_params,
    out_type=(
        jax.ShapeDtypeStruct((32 * ESTRIDE,), jnp.int32),
        jax.ShapeDtypeStruct((32 * ESTRIDE,), jnp.int32),
        jax.ShapeDtypeStruct((32, 16), jnp.int32),
    ),
    scratch_types=[
        pltpu.VMEM((EC,), jnp.int32),
        pltpu.VMEM((EC,), jnp.int32),
        pltpu.VMEM((EC,), jnp.int32),
        pltpu.VMEM((RING,), jnp.int32),
        pltpu.VMEM((RING,), jnp.int32),
        pltpu.VMEM((16,), jnp.int32),
        pltpu.SemaphoreType.DMA,
        pltpu.SemaphoreType.DMA,
        pltpu.SemaphoreType.DMA,
    ],
)
def _sc_route(src_hbm, dst_hbm, kk_hbm, flat_hbm, loc_hbm, cnt_hbm,
              srcv, dstv, k_v, rflat, rloc, cbuf, semx, semy, semz):
    c = lax.axis_index("c")
    s = lax.axis_index("s")
    tid = s * 2 + c
    lbase = tid * ESTRIDE

    zero16 = jnp.zeros((16,), jnp.int32)

    def init_body(i, carry):
        rflat[pl.ds(i * 16, 16)] = zero16
        rloc[pl.ds(i * 16, 16)] = zero16
        return carry

    lax.fori_loop(0, RING // 16, init_body, jnp.int32(0))

    lo_v = jnp.full((16,), tid * RPT, jnp.int32)
    rpt_v = jnp.full((16,), RPT, jnp.int32)
    np_v = jnp.full((16,), NP_, jnp.int32)
    one_v = jnp.full((16,), 1, jnp.int32)
    rmask_v = jnp.full((16,), RING - 1, jnp.int32)

    idx15 = jnp.full((16,), 15, jnp.int32)

    def comp_body(i, carry):
        cnt_v, fl = carry
        ga = []
        for h in range(2):
            dv = dstv[pl.ds(i * 32 + h * 16, 16)]
            sv = srcv[pl.ds(i * 32 + h * 16, 16)]
            kv = k_v[pl.ds(i * 32 + h * 16, 16)]
            loc = dv - lo_v
            m = (loc >= zero16) & (loc < rpt_v)
            flat = kv * np_v + sv
            mi = m.astype(jnp.int32)
            scan = plsc.cumsum(mi)
            ga.append((loc, m, flat, scan))
        loc0, m0, flat0, scan0 = ga[0]
        loc1, m1, flat1, scan1 = ga[1]
        tot0 = lax.gather(
            scan0, idx15[:, None],
            lax.GatherDimensionNumbers(offset_dims=(),
                                       collapsed_slice_dims=(0,),
                                       start_index_map=(0,)),
            (1,), mode=lax.GatherScatterMode.PROMISE_IN_BOUNDS)
        tot1 = lax.gather(
            scan1, idx15[:, None],
            lax.GatherDimensionNumbers(offset_dims=(),
                                       collapsed_slice_dims=(0,),
                                       start_index_map=(0,)),
            (1,), mode=lax.GatherScatterMode.PROMISE_IN_BOUNDS)
        pos0 = (cnt_v + scan0 - one_v) & rmask_v
        base1 = cnt_v + tot0
        pos1 = (base1 + scan1 - one_v) & rmask_v
        plsc.store_scatter(rflat, [pos0], flat0, mask=m0)
        plsc.store_scatter(rloc, [pos0], loc0, mask=m0)
        plsc.store_scatter(rflat, [pos1], flat1, mask=m1)
        plsc.store_scatter(rloc, [pos1], loc1, mask=m1)
        return base1 + tot1, fl

    def flush_body(fi, fl):
        rpos = pl.multiple_of(fl % RING, FB)
        fl8 = pl.multiple_of(lbase + fl, FB)
        pltpu.sync_copy(rflat.at[pl.ds(rpos, FB)],
                        flat_hbm.at[pl.ds(fl8, FB)])
        pltpu.sync_copy(rloc.at[pl.ds(rpos, FB)],
                        loc_hbm.at[pl.ds(fl8, FB)])
        return fl + FB

    def chunk_body(ci, carry):
        cnt_v, fl = carry
        cbase = pl.multiple_of(ci * EC, 8)
        d1 = pltpu.async_copy(src_hbm.at[pl.ds(cbase, EC)], srcv, semx)
        d2 = pltpu.async_copy(dst_hbm.at[pl.ds(cbase, EC)], dstv, semy)
        d3 = pltpu.async_copy(kk_hbm.at[pl.ds(cbase, EC)], k_v, semz)
        d1.wait()
        d2.wait()
        d3.wait()
        cnt_v, fl = lax.fori_loop(0, EC // 32, comp_body, (cnt_v, fl))
        nf = (jnp.max(cnt_v) - fl) // FB
        fl = lax.fori_loop(0, nf, flush_body, fl)
        return cnt_v, fl

    cnt_v, fl = lax.fori_loop(0, NCH, chunk_body,
                              (jnp.zeros((16,), jnp.int32), jnp.int32(0)))
    cnt = jnp.max(cnt_v)
    for _ in range(4):
        fl = flush_body(0, fl)

    cbuf[...] = jnp.full((16,), cnt, jnp.int32)
    pltpu.sync_copy(cbuf, cnt_hbm.at[tid])


@functools.partial(
    pl.kernel,
    mesh=_sc_mesh,
    compiler_params=_sc_params,
    out_type=jax.ShapeDtypeStruct((NP_, CH), jnp.float32),
    scratch_types=[
        pltpu.VMEM((RPT, CH), jnp.float32),
        pltpu.VMEM((GB, CH), jnp.float32),
        pltpu.VMEM((GB, CH), jnp.float32),
        pltpu.VMEM((16 * GB,), jnp.int32),
        pltpu.VMEM((16 * GB,), jnp.int32),
        pltpu.VMEM((16,), jnp.int32),
        pltpu.SemaphoreType.DMA,
        pltpu.SemaphoreType.DMA,
    ],
)
def _sc_conv(h_tab, flat_hbm, loc_hbm, cnt_hbm, zero_hbm, out_hbm,
             acc, stage0, stage1, lflat, lloc, cbuf, sem0, sem1):
    c = lax.axis_index("c")
    s = lax.axis_index("s")
    tid = s * 2 + c
    lbase = tid * ESTRIDE
    lchunk = 16 * GB

    pltpu.sync_copy(zero_hbm, acc)
    pltpu.sync_copy(cnt_hbm.at[tid], cbuf)
    cnt = jnp.max(cbuf[...])
    nb = (cnt + GB - 1) // GB
    nc = (nb + 15) // 16

    iota16 = lax.iota(jnp.int32, 16)
    colb = [iota16 + jnp.full((16,), g * 16, jnp.int32)
            for g in range(CH // 16)]
    _dn = lax.GatherDimensionNumbers(offset_dims=(),
                                     collapsed_slice_dims=(0,),
                                     start_index_map=(0,))

    def chunk_body(ch, carry):
        chb = pl.multiple_of(lbase + ch * lchunk, 8)
        dl1 = pltpu.async_copy(flat_hbm.at[pl.ds(chb, lchunk)], lflat, sem0)
        dl2 = pltpu.async_copy(loc_hbm.at[pl.ds(chb, lchunk)], lloc, sem1)
        dl1.wait()
        dl2.wait()
        base_b = ch * 16

        def issue(jj, st, sm):
            cond = base_b + jj < nb

            @pl.when(cond)
            def _():
                pltpu.async_copy(h_tab.at[lflat.at[pl.ds(jj * GB, GB)]],
                                 st, sm)

            return cond

        def drain(jj, st):
            gj = base_b + jj
            nvalid = jnp.clip(cnt - gj * GB, 0, GB)

            def one(r):
                lv = lloc[pl.ds(jj * GB + (r // 16) * 16, 16)]
                lane_v = jnp.full((16,), r % 16, jnp.int32)
                locsplat = lax.gather(
                    lv, lane_v[:, None], _dn, (1,),
                    mode=lax.GatherScatterMode.PROMISE_IN_BOUNDS)
                for g in range(CH // 16):
                    plsc.addupdate_scatter(acc, [locsplat, colb[g]],
                                           st[r, pl.ds(g * 16, 16)])

            def dbody2(q, cc):
                one(q * 2)
                one(q * 2 + 1)
                return cc

            lax.fori_loop(0, nvalid // 2, dbody2, jnp.int32(0))

            @pl.when(nvalid % 2 == 1)
            def _():
                one(nvalid - 1)

        slots = [(stage0, sem0), (stage1, sem1)]
        conds = {0: issue(0, *slots[0])}
        for jj in range(16):
            st, sm = slots[jj % 2]
            if jj + 1 < 16:
                conds[jj + 1] = issue(jj + 1, *slots[(jj + 1) % 2])

            @pl.when(conds[jj])
            def _(st=st, sm=sm):
                pltpu.make_async_copy(h_tab.at[pl.ds(0, GB)], st, sm).wait()

            drain(jj, st)
        return carry

    lax.fori_loop(0, nc, chunk_body, jnp.int32(0))

    pltpu.sync_copy(acc, out_hbm.at[pl.ds(tid * RPT, RPT)])


def kernel(x, edge_index, edge_kernel, W1, g1, b1, W2, g2, b2):
    src = edge_index[0].astype(jnp.int32)
    dst = edge_index[1].astype(jnp.int32)
    kk = edge_kernel.astype(jnp.int32)
    x_pad = jnp.pad(x, ((0, NP_ - NN), (0, 0)))
    W1b = W1.astype(jnp.bfloat16)
    W2b = W2.astype(jnp.bfloat16)
    zeros = jnp.zeros((RPT, CH), jnp.float32)
    g1r = g1.reshape(1, CH)
    b1r = b1.reshape(1, CH)
    g2r = g2.reshape(1, CH)
    b2r = b2.reshape(1, CH)

    flat_l, loc_l, cnts = _sc_route(src, dst, kk)

    H1 = _mm27(x_pad, W1b).reshape(KV * NP_, CH)
    h1 = _sc_conv(H1, flat_l, loc_l, cnts, zeros)
    H2 = _ln_relu_mm27(h1, g1r, b1r, W2b).reshape(KV * NP_, CH)
    h2 = _sc_conv(H2, flat_l, loc_l, cnts, zeros)
    return _final(h2[:NN], g2r, b2r, x)
